# baseline (speedup 1.0000x reference)
import jax
import jax.numpy as jnp
from jax import lax
from jax.experimental import pallas as pl
from jax.experimental.pallas import tpu as pltpu
from jax.experimental.pallas import tpu_sc as plsc

N, B, D = 16384, 512, 1024
C_FLAT, C_GLOB = 4096, 4096
NW = 32
SENT_PER_W = N // NW
CHUNKS = SENT_PER_W // 16
ITEMS = 3 * CHUNKS
BAGS_PER_W = B // NW
NEG = -3.4e38

_SC_PARAMS = pltpu.CompilerParams(use_tc_tiling_on_sc=False,
                                  needs_layout_passes=False)


def _wid():
    return lax.axis_index("s") * 2 + lax.axis_index("c")


def _iota16():
    return lax.iota(jnp.int32, 16)


CH = 8
ROWS = 3 * CH
NCH = SENT_PER_W // CH


def _logits_body(x_flat, q3n, aw, out2, qall, pbuf,
                 ai0, ai1, xb0, xb1, ab0, ab1,
                 sx0, sx1, sa0, sa1):
    w = _wid()
    base = w * SENT_PER_W
    it = _iota16()
    pltpu.sync_copy(q3n.at[pl.ds(base * 3, 3 * SENT_PER_W)], qall)

    slots = ((ai0, xb0, ab0, sx0, sa0), (ai1, xb1, ab1, sx1, sa1))

    def issue(c, slot):
        ai, xb, ab, sx, sa = slot
        row0 = base * 3 + c * ROWS
        it16 = _iota16()
        ai[pl.ds(0, 16)] = plsc.load_gather(qall, [c * ROWS + it16])
        plsc.store_scatter(ai, [8 + it16],
                           plsc.load_gather(qall, [c * ROWS + 8 + it16]))
        pltpu.async_copy(x_flat.at[pl.ds(row0 * D, ROWS * D)], xb, sx)
        pltpu.async_copy(aw.at[ai], ab, sa)

    def drain(c, slot):
        ai, xb, ab, sx, sa = slot
        row0 = base * 3 + c * ROWS
        pltpu.make_async_copy(x_flat.at[pl.ds(row0 * D, ROWS * D)], xb, sx).wait()
        pltpu.make_async_copy(aw.at[ai], ab, sa).wait()

    def compute(c, slot):
        ai, xb, ab, sx, sa = slot

        def row_loop(r, carry):
            va, vb = carry
            rb = r * D

            def jstep(j, acc):
                o = j * 16
                return acc + xb[pl.ds(rb + o, 16)] * ab[r, pl.ds(o, 16)]

            acc = lax.fori_loop(0, D // 16, jstep, jnp.zeros((16,), jnp.float32))
            t = jnp.sum(acc)
            va = jnp.where(it == r, t, va)
            vb = jnp.where(it == r - 16, t, vb)
            return va, vb

        z = jnp.zeros((16,), jnp.float32)
        va, vb = lax.fori_loop(0, ROWS, row_loop, (z, z))
        ra = c * ROWS + it
        pa = (ra - (ra // 3) * 3) * SENT_PER_W + ra // 3
        plsc.store_scatter(pbuf, [pa], va)
        rb2 = c * ROWS + 16 + it
        pb2 = (rb2 - (rb2 // 3) * 3) * SENT_PER_W + rb2 // 3
        plsc.store_scatter(pbuf, [pb2], vb, mask=it < 8)

    issue(0, slots[0])
    issue(1, slots[1])

    def pair_body(c2, carry):
        c = c2 * 2
        for k in range(2):
            slot = slots[k]
            drain(c + k, slot)
            compute(c + k, slot)

            @pl.when(c + k + 2 < NCH)
            def _():
                issue(c + k + 2, slot)
        return carry

    lax.fori_loop(0, NCH // 2, pair_body, 0)
    for l in range(3):
        pltpu.sync_copy(pbuf.at[pl.ds(l * SENT_PER_W, SENT_PER_W)],
                        out2.at[pl.ds(l * N + base, SENT_PER_W)])


def _sc_logits(x_flat, q3n, aw):
    kfn = pl.kernel(
        _logits_body,
        out_type=jax.ShapeDtypeStruct((3 * N,), jnp.float32),
        mesh=plsc.VectorSubcoreMesh(core_axis_name="c", subcore_axis_name="s"),
        compiler_params=_SC_PARAMS,
        scratch_types=[
            pltpu.VMEM((3 * SENT_PER_W,), jnp.int32),
            pltpu.VMEM((3 * SENT_PER_W,), jnp.float32),
            pltpu.VMEM((ROWS,), jnp.int32),
            pltpu.VMEM((ROWS,), jnp.int32),
            pltpu.VMEM((ROWS * D,), jnp.float32),
            pltpu.VMEM((ROWS * D,), jnp.float32),
            pltpu.VMEM((ROWS, D), jnp.float32),
            pltpu.VMEM((ROWS, D), jnp.float32),
            pltpu.SemaphoreType.DMA,
            pltpu.SemaphoreType.DMA,
            pltpu.SemaphoreType.DMA,
            pltpu.SemaphoreType.DMA,
        ],
    )
    return kfn(x_flat, q3n, aw)


def _stats_body(lg, scope_pad, m_out, s_out, lgv, scv, mbuf, sbuf):
    w = _wid()
    it = _iota16()
    pltpu.sync_copy(scope_pad, scv)
    pltpu.sync_copy(lg, lgv)
    lo_vec = plsc.load_gather(scv, [w * BAGS_PER_W + it])
    hi_vec = plsc.load_gather(scv, [w * BAGS_PER_W + 1 + it])
    for l in range(3):
        mrow = jnp.full((16,), NEG, jnp.float32)
        srow = jnp.zeros((16,), jnp.float32)
        for b in range(BAGS_PER_W):
            sel = it == b
            start = jnp.max(jnp.where(sel, lo_vec, -2147483647))
            end = jnp.max(jnp.where(sel, hi_vec, -2147483647))
            nch = (end - start + 15) // 16

            def max_step(ci, acc):
                idx = start + ci * 16 + it
                v = plsc.load_gather(lgv, [l * N + jnp.minimum(idx, N - 1)])
                return jnp.maximum(acc, jnp.where(idx < end, v, NEG))

            mvec = lax.fori_loop(0, nch, max_step, jnp.full((16,), NEG, jnp.float32))
            m = jnp.max(mvec)

            def sum_step(ci, acc):
                idx = start + ci * 16 + it
                v = plsc.load_gather(lgv, [l * N + jnp.minimum(idx, N - 1)])
                return acc + jnp.where(idx < end, jnp.exp(v - m), 0.0)

            svec = lax.fori_loop(0, nch, sum_step, jnp.zeros((16,), jnp.float32))
            s = jnp.sum(svec)
            mrow = jnp.where(sel, m, mrow)
            srow = jnp.where(sel, s, srow)
        mbuf[...] = mrow
        sbuf[...] = srow
        pltpu.sync_copy(mbuf, m_out.at[pl.ds(l * B + w * BAGS_PER_W, BAGS_PER_W)])
        pltpu.sync_copy(sbuf, s_out.at[pl.ds(l * B + w * BAGS_PER_W, BAGS_PER_W)])


def _sc_stats(lg, scope_pad):
    kfn = pl.kernel(
        _stats_body,
        out_type=(
            jax.ShapeDtypeStruct((3 * B,), jnp.float32),
            jax.ShapeDtypeStruct((3 * B,), jnp.float32),
        ),
        mesh=plsc.VectorSubcoreMesh(core_axis_name="c", subcore_axis_name="s"),
        compiler_params=_SC_PARAMS,
        scratch_types=[
            pltpu.VMEM((3 * N,), jnp.float32),
            pltpu.VMEM((520,), jnp.int32),
            pltpu.VMEM((16,), jnp.float32),
            pltpu.VMEM((16,), jnp.float32),
        ],
    )
    return kfn(lg, scope_pad)


SB = 512


def _pool_body(x_ref, lg_ref, m_ref, s_ref, lo_ref, hi_ref, out_ref, cat_ref):
    sb = pl.program_id(0)
    rows = sb * SB + lax.broadcasted_iota(jnp.int32, (1, SB), 1)
    lo = lo_ref[...]
    hi = hi_ref[...]
    mask = (rows >= lo) & (rows < hi)
    for l in range(3):
        lg = lg_ref[l:l + 1, :]
        m = m_ref[l]
        s = s_ref[l]
        e = jnp.where(mask, jnp.exp(lg - m), 0.0)
        wmat = (e / jnp.maximum(s, 1e-20)).astype(jnp.bfloat16)
        xb = x_ref[:, l * D:(l + 1) * D].astype(jnp.bfloat16)
        contrib = lax.dot_general(wmat, xb, (((1,), (0,)), ((), ())),
                                  preferred_element_type=jnp.float32)

        @pl.when(sb == 0)
        def _():
            out_ref[l] = contrib

        @pl.when(sb != 0)
        def _():
            out_ref[l] = out_ref[l] + contrib

        @pl.when(sb == N // SB - 1)
        def _():
            cat_ref[:, l * D:(l + 1) * D] = out_ref[l]


def _tc_pool(x2, lg2, m3, s3, lo2, hi2):
    return pl.pallas_call(
        _pool_body,
        grid=(N // SB,),
        in_specs=[
            pl.BlockSpec((SB, 3 * D), lambda sb: (sb, 0)),
            pl.BlockSpec((3, SB), lambda sb: (0, sb)),
            pl.BlockSpec((3, B, 1), lambda sb: (0, 0, 0)),
            pl.BlockSpec((3, B, 1), lambda sb: (0, 0, 0)),
            pl.BlockSpec((B, 1), lambda sb: (0, 0)),
            pl.BlockSpec((B, 1), lambda sb: (0, 0)),
        ],
        out_specs=[
            pl.BlockSpec((3, B, D), lambda sb: (0, 0, 0)),
            pl.BlockSpec((B, 3 * D), lambda sb: (0, 0)),
        ],
        out_shape=[
            jax.ShapeDtypeStruct((3, B, D), jnp.float32),
            jax.ShapeDtypeStruct((B, 3 * D), jnp.float32),
        ],
    )(x2, lg2, m3, s3, lo2, hi2)


CBLK = 512


def _mm_body(lt_ref, wr_ref, b_ref, out_ref):
    acc = lax.dot_general(lt_ref[...], wr_ref[...], (((1,), (1,)), ((), ())),
                          preferred_element_type=jnp.float32)
    out_ref[...] = acc + b_ref[...]


def _tc_matmul(lt, wr, bias2):
    return pl.pallas_call(
        _mm_body,
        grid=(C_FLAT // CBLK,),
        in_specs=[
            pl.BlockSpec((B, 3 * D), lambda c: (0, 0)),
            pl.BlockSpec((CBLK, 3 * D), lambda c: (c, 0)),
            pl.BlockSpec((1, CBLK), lambda c: (0, c)),
        ],
        out_specs=pl.BlockSpec((B, CBLK), lambda c: (0, c)),
        out_shape=jax.ShapeDtypeStruct((B, C_FLAT), jnp.float32),
    )(lt.astype(jnp.bfloat16), wr.astype(jnp.bfloat16), bias2)


def kernel(x, attention_query, scope, relation_weight, bias, attention_weight):
    x_flat = x.reshape(N * 3 * D)
    q3n = attention_query.astype(jnp.int32).reshape(3 * N)
    scope_pad = jnp.pad(scope.astype(jnp.int32), (0, 520 - B - 1))

    lg2 = _sc_logits(x_flat, q3n, attention_weight)
    m3, s3 = _sc_stats(lg2, scope_pad)

    lo2 = scope[:B].astype(jnp.int32).reshape(B, 1)
    hi2 = scope[1:].astype(jnp.int32).reshape(B, 1)
    layers, logits_total = _tc_pool(
        x.reshape(N, 3 * D), lg2.reshape(3, N), m3.reshape(3, B, 1),
        s3.reshape(3, B, 1), lo2, hi2)

    probs = _tc_matmul(logits_total, relation_weight, bias.reshape(1, C_FLAT))
    return (layers, logits_total, probs)


# speedup vs baseline: 1.2232x; 1.2232x over previous
import jax
import jax.numpy as jnp
from jax import lax
from jax.experimental import pallas as pl
from jax.experimental.pallas import tpu as pltpu
from jax.experimental.pallas import tpu_sc as plsc

N, B, D = 16384, 512, 1024
C_FLAT, C_GLOB = 4096, 4096
NW = 32
SENT_PER_W = N // NW
CHUNKS = SENT_PER_W // 16
ITEMS = 3 * CHUNKS
BAGS_PER_W = B // NW
NEG = -3.4e38

_SC_PARAMS = pltpu.CompilerParams(use_tc_tiling_on_sc=False,
                                  needs_layout_passes=False)


def _wid():
    return lax.axis_index("s") * 2 + lax.axis_index("c")


def _iota16():
    return lax.iota(jnp.int32, 16)


CH = 8
ROWS = 3 * CH
NCH = SENT_PER_W // CH


def _logits_body(x_flat, q3n, aw, out2, qall, pbuf,
                 ai0, ai1, xb0, xb1, ab0, ab1,
                 sx0, sx1, sa0, sa1):
    w = _wid()
    base = w * SENT_PER_W
    it = _iota16()
    pltpu.sync_copy(q3n.at[pl.ds(base * 3, 3 * SENT_PER_W)], qall)

    slots = ((ai0, xb0, ab0, sx0, sa0), (ai1, xb1, ab1, sx1, sa1))

    def issue(c, slot):
        ai, xb, ab, sx, sa = slot
        row0 = base * 3 + c * ROWS
        it16 = _iota16()
        ai[pl.ds(0, 16)] = plsc.load_gather(qall, [c * ROWS + it16])
        plsc.store_scatter(ai, [8 + it16],
                           plsc.load_gather(qall, [c * ROWS + 8 + it16]))
        pltpu.async_copy(x_flat.at[pl.ds(row0 * D, ROWS * D)], xb, sx)
        pltpu.async_copy(aw.at[ai], ab, sa)

    def drain(c, slot):
        ai, xb, ab, sx, sa = slot
        row0 = base * 3 + c * ROWS
        pltpu.make_async_copy(x_flat.at[pl.ds(row0 * D, ROWS * D)], xb, sx).wait()
        pltpu.make_async_copy(aw.at[ai], ab, sa).wait()

    def compute(c, slot):
        ai, xb, ab, sx, sa = slot

        def row_loop(r, carry):
            va, vb = carry
            rb = r * D

            def jstep(j, acc):
                for u in range(4):
                    o = (j * 4 + u) * 16
                    acc = acc + xb[pl.ds(rb + o, 16)] * ab[r, pl.ds(o, 16)]
                return acc

            acc = lax.fori_loop(0, D // 64, jstep, jnp.zeros((16,), jnp.float32))
            t = jnp.sum(acc)
            va = jnp.where(it == r, t, va)
            vb = jnp.where(it == r - 16, t, vb)
            return va, vb

        z = jnp.zeros((16,), jnp.float32)
        va, vb = lax.fori_loop(0, ROWS, row_loop, (z, z))
        ra = c * ROWS + it
        pa = (ra - (ra // 3) * 3) * SENT_PER_W + ra // 3
        plsc.store_scatter(pbuf, [pa], va)
        rb2 = c * ROWS + 16 + it
        pb2 = (rb2 - (rb2 // 3) * 3) * SENT_PER_W + rb2 // 3
        plsc.store_scatter(pbuf, [pb2], vb, mask=it < 8)

    issue(0, slots[0])
    issue(1, slots[1])

    def pair_body(c2, carry):
        c = c2 * 2
        for k in range(2):
            slot = slots[k]
            drain(c + k, slot)
            compute(c + k, slot)

            @pl.when(c + k + 2 < NCH)
            def _():
                issue(c + k + 2, slot)
        return carry

    lax.fori_loop(0, NCH // 2, pair_body, 0)
    for l in range(3):
        pltpu.sync_copy(pbuf.at[pl.ds(l * SENT_PER_W, SENT_PER_W)],
                        out2.at[pl.ds(l * N + base, SENT_PER_W)])


def _sc_logits(x_flat, q3n, aw):
    kfn = pl.kernel(
        _logits_body,
        out_type=jax.ShapeDtypeStruct((3 * N,), jnp.float32),
        mesh=plsc.VectorSubcoreMesh(core_axis_name="c", subcore_axis_name="s"),
        compiler_params=_SC_PARAMS,
        scratch_types=[
            pltpu.VMEM((3 * SENT_PER_W,), jnp.int32),
            pltpu.VMEM((3 * SENT_PER_W,), jnp.float32),
            pltpu.VMEM((ROWS,), jnp.int32),
            pltpu.VMEM((ROWS,), jnp.int32),
            pltpu.VMEM((ROWS * D,), jnp.float32),
            pltpu.VMEM((ROWS * D,), jnp.float32),
            pltpu.VMEM((ROWS, D), jnp.float32),
            pltpu.VMEM((ROWS, D), jnp.float32),
            pltpu.SemaphoreType.DMA,
            pltpu.SemaphoreType.DMA,
            pltpu.SemaphoreType.DMA,
            pltpu.SemaphoreType.DMA,
        ],
    )
    return kfn(x_flat, q3n, aw)


def _stats_body(lg, scope_pad, m_out, s_out, lgv, scv, mbuf, sbuf):
    w = _wid()
    it = _iota16()
    pltpu.sync_copy(scope_pad, scv)
    pltpu.sync_copy(lg, lgv)
    lo_vec = plsc.load_gather(scv, [w * BAGS_PER_W + it])
    hi_vec = plsc.load_gather(scv, [w * BAGS_PER_W + 1 + it])
    for l in range(3):
        mrow = jnp.full((16,), NEG, jnp.float32)
        srow = jnp.zeros((16,), jnp.float32)
        for b in range(BAGS_PER_W):
            sel = it == b
            start = jnp.max(jnp.where(sel, lo_vec, -2147483647))
            end = jnp.max(jnp.where(sel, hi_vec, -2147483647))
            nch = (end - start + 15) // 16

            def max_step(ci, acc):
                idx = start + ci * 16 + it
                v = plsc.load_gather(lgv, [l * N + jnp.minimum(idx, N - 1)])
                return jnp.maximum(acc, jnp.where(idx < end, v, NEG))

            mvec = lax.fori_loop(0, nch, max_step, jnp.full((16,), NEG, jnp.float32))
            m = jnp.max(mvec)

            def sum_step(ci, acc):
                idx = start + ci * 16 + it
                v = plsc.load_gather(lgv, [l * N + jnp.minimum(idx, N - 1)])
                return acc + jnp.where(idx < end, jnp.exp(v - m), 0.0)

            svec = lax.fori_loop(0, nch, sum_step, jnp.zeros((16,), jnp.float32))
            s = jnp.sum(svec)
            mrow = jnp.where(sel, m, mrow)
            srow = jnp.where(sel, s, srow)
        mbuf[...] = mrow
        sbuf[...] = srow
        pltpu.sync_copy(mbuf, m_out.at[pl.ds(l * B + w * BAGS_PER_W, BAGS_PER_W)])
        pltpu.sync_copy(sbuf, s_out.at[pl.ds(l * B + w * BAGS_PER_W, BAGS_PER_W)])


def _sc_stats(lg, scope_pad):
    kfn = pl.kernel(
        _stats_body,
        out_type=(
            jax.ShapeDtypeStruct((3 * B,), jnp.float32),
            jax.ShapeDtypeStruct((3 * B,), jnp.float32),
        ),
        mesh=plsc.VectorSubcoreMesh(core_axis_name="c", subcore_axis_name="s"),
        compiler_params=_SC_PARAMS,
        scratch_types=[
            pltpu.VMEM((3 * N,), jnp.float32),
            pltpu.VMEM((520,), jnp.int32),
            pltpu.VMEM((16,), jnp.float32),
            pltpu.VMEM((16,), jnp.float32),
        ],
    )
    return kfn(lg, scope_pad)


SB = 512


def _pool_body(x_ref, lg_ref, m_ref, s_ref, lo_ref, hi_ref, out_ref, cat_ref):
    sb = pl.program_id(0)
    rows = sb * SB + lax.broadcasted_iota(jnp.int32, (1, SB), 1)
    lo = lo_ref[...]
    hi = hi_ref[...]
    mask = (rows >= lo) & (rows < hi)
    for l in range(3):
        lg = lg_ref[l:l + 1, :]
        m = m_ref[l]
        s = s_ref[l]
        e = jnp.where(mask, jnp.exp(lg - m), 0.0)
        wmat = (e / jnp.maximum(s, 1e-20)).astype(jnp.bfloat16)
        xb = x_ref[:, l * D:(l + 1) * D].astype(jnp.bfloat16)
        contrib = lax.dot_general(wmat, xb, (((1,), (0,)), ((), ())),
                                  preferred_element_type=jnp.float32)

        @pl.when(sb == 0)
        def _():
            out_ref[l] = contrib

        @pl.when(sb != 0)
        def _():
            out_ref[l] = out_ref[l] + contrib

        @pl.when(sb == N // SB - 1)
        def _():
            cat_ref[:, l * D:(l + 1) * D] = out_ref[l]


def _tc_pool(x2, lg2, m3, s3, lo2, hi2):
    return pl.pallas_call(
        _pool_body,
        grid=(N // SB,),
        in_specs=[
            pl.BlockSpec((SB, 3 * D), lambda sb: (sb, 0)),
            pl.BlockSpec((3, SB), lambda sb: (0, sb)),
            pl.BlockSpec((3, B, 1), lambda sb: (0, 0, 0)),
            pl.BlockSpec((3, B, 1), lambda sb: (0, 0, 0)),
            pl.BlockSpec((B, 1), lambda sb: (0, 0)),
            pl.BlockSpec((B, 1), lambda sb: (0, 0)),
        ],
        out_specs=[
            pl.BlockSpec((3, B, D), lambda sb: (0, 0, 0)),
            pl.BlockSpec((B, 3 * D), lambda sb: (0, 0)),
        ],
        out_shape=[
            jax.ShapeDtypeStruct((3, B, D), jnp.float32),
            jax.ShapeDtypeStruct((B, 3 * D), jnp.float32),
        ],
    )(x2, lg2, m3, s3, lo2, hi2)


CBLK = 512


def _mm_body(lt_ref, wr_ref, b_ref, out_ref):
    acc = lax.dot_general(lt_ref[...], wr_ref[...], (((1,), (1,)), ((), ())),
                          preferred_element_type=jnp.float32)
    out_ref[...] = acc + b_ref[...]


def _tc_matmul(lt, wr, bias2):
    return pl.pallas_call(
        _mm_body,
        grid=(C_FLAT // CBLK,),
        in_specs=[
            pl.BlockSpec((B, 3 * D), lambda c: (0, 0)),
            pl.BlockSpec((CBLK, 3 * D), lambda c: (c, 0)),
            pl.BlockSpec((1, CBLK), lambda c: (0, c)),
        ],
        out_specs=pl.BlockSpec((B, CBLK), lambda c: (0, c)),
        out_shape=jax.ShapeDtypeStruct((B, C_FLAT), jnp.float32),
    )(lt.astype(jnp.bfloat16), wr.astype(jnp.bfloat16), bias2)


def kernel(x, attention_query, scope, relation_weight, bias, attention_weight):
    x_flat = x.reshape(N * 3 * D)
    q3n = attention_query.astype(jnp.int32).reshape(3 * N)
    scope_pad = jnp.pad(scope.astype(jnp.int32), (0, 520 - B - 1))

    lg2 = _sc_logits(x_flat, q3n, attention_weight)
    m3, s3 = _sc_stats(lg2, scope_pad)

    lo2 = scope[:B].astype(jnp.int32).reshape(B, 1)
    hi2 = scope[1:].astype(jnp.int32).reshape(B, 1)
    layers, logits_total = _tc_pool(
        x.reshape(N, 3 * D), lg2.reshape(3, N), m3.reshape(3, B, 1),
        s3.reshape(3, B, 1), lo2, hi2)

    probs = _tc_matmul(logits_total, relation_weight, bias.reshape(1, C_FLAT))
    return (layers, logits_total, probs)


# speedup vs baseline: 1.2451x; 1.0179x over previous
import jax
import jax.numpy as jnp
from jax import lax
from jax.experimental import pallas as pl
from jax.experimental.pallas import tpu as pltpu
from jax.experimental.pallas import tpu_sc as plsc

N, B, D = 16384, 512, 1024
C_FLAT, C_GLOB = 4096, 4096
NW = 32
SENT_PER_W = N // NW
CHUNKS = SENT_PER_W // 16
ITEMS = 3 * CHUNKS
BAGS_PER_W = B // NW
NEG = -3.4e38

_SC_PARAMS = pltpu.CompilerParams(use_tc_tiling_on_sc=False,
                                  needs_layout_passes=False)


def _wid():
    return lax.axis_index("s") * 2 + lax.axis_index("c")


def _iota16():
    return lax.iota(jnp.int32, 16)


CH = 8
ROWS = 3 * CH
NCH = SENT_PER_W // CH


def _logits_body(x_flat, q3n, aw, out2, qall, pbuf,
                 ai0, ai1, xb0, xb1, ab0, ab1,
                 sx0, sx1, sa0, sa1):
    w = _wid()
    base = w * SENT_PER_W
    it = _iota16()
    pltpu.sync_copy(q3n.at[pl.ds(base * 3, 3 * SENT_PER_W)], qall)

    slots = ((ai0, xb0, ab0, sx0, sa0), (ai1, xb1, ab1, sx1, sa1))

    def issue(c, slot):
        ai, xb, ab, sx, sa = slot
        row0 = base * 3 + c * ROWS
        it16 = _iota16()
        ai[pl.ds(0, 16)] = plsc.load_gather(qall, [c * ROWS + it16])
        plsc.store_scatter(ai, [8 + it16],
                           plsc.load_gather(qall, [c * ROWS + 8 + it16]))
        pltpu.async_copy(x_flat.at[pl.ds(row0 * D, ROWS * D)], xb, sx)
        pltpu.async_copy(aw.at[ai], ab, sa)

    def drain(c, slot):
        ai, xb, ab, sx, sa = slot
        row0 = base * 3 + c * ROWS
        pltpu.make_async_copy(x_flat.at[pl.ds(row0 * D, ROWS * D)], xb, sx).wait()
        pltpu.make_async_copy(aw.at[ai], ab, sa).wait()

    def compute(c, slot):
        ai, xb, ab, sx, sa = slot

        def row_loop(r, carry):
            va, vb = carry
            rb = r * D

            def jstep(j, acc):
                for u in range(4):
                    o = (j * 4 + u) * 16
                    acc = acc + xb[pl.ds(rb + o, 16)] * ab[r, pl.ds(o, 16)]
                return acc

            acc = lax.fori_loop(0, D // 64, jstep, jnp.zeros((16,), jnp.float32))
            t = jnp.sum(acc)
            va = jnp.where(it == r, t, va)
            vb = jnp.where(it == r - 16, t, vb)
            return va, vb

        z = jnp.zeros((16,), jnp.float32)
        va, vb = lax.fori_loop(0, ROWS, row_loop, (z, z))
        ra = c * ROWS + it
        pa = (ra - (ra // 3) * 3) * SENT_PER_W + ra // 3
        plsc.store_scatter(pbuf, [pa], va)
        rb2 = c * ROWS + 16 + it
        pb2 = (rb2 - (rb2 // 3) * 3) * SENT_PER_W + rb2 // 3
        plsc.store_scatter(pbuf, [pb2], vb, mask=it < 8)

    issue(0, slots[0])
    issue(1, slots[1])

    def pair_body(c2, carry):
        c = c2 * 2
        for k in range(2):
            slot = slots[k]
            drain(c + k, slot)
            compute(c + k, slot)

            @pl.when(c + k + 2 < NCH)
            def _():
                issue(c + k + 2, slot)
        return carry

    lax.fori_loop(0, NCH // 2, pair_body, 0)
    for l in range(3):
        pltpu.sync_copy(pbuf.at[pl.ds(l * SENT_PER_W, SENT_PER_W)],
                        out2.at[pl.ds(l * N + base, SENT_PER_W)])


def _sc_logits(x_flat, q3n, aw):
    kfn = pl.kernel(
        _logits_body,
        out_type=jax.ShapeDtypeStruct((3 * N,), jnp.float32),
        mesh=plsc.VectorSubcoreMesh(core_axis_name="c", subcore_axis_name="s"),
        compiler_params=_SC_PARAMS,
        scratch_types=[
            pltpu.VMEM((3 * SENT_PER_W,), jnp.int32),
            pltpu.VMEM((3 * SENT_PER_W,), jnp.float32),
            pltpu.VMEM((ROWS,), jnp.int32),
            pltpu.VMEM((ROWS,), jnp.int32),
            pltpu.VMEM((ROWS * D,), jnp.float32),
            pltpu.VMEM((ROWS * D,), jnp.float32),
            pltpu.VMEM((ROWS, D), jnp.float32),
            pltpu.VMEM((ROWS, D), jnp.float32),
            pltpu.SemaphoreType.DMA,
            pltpu.SemaphoreType.DMA,
            pltpu.SemaphoreType.DMA,
            pltpu.SemaphoreType.DMA,
        ],
    )
    return kfn(x_flat, q3n, aw)


def _stats_body(lg, scope_pad, m_out, s_out, lgv, scv, mbuf, sbuf):
    w = _wid()
    it = _iota16()
    pltpu.sync_copy(scope_pad, scv)
    pltpu.sync_copy(lg, lgv)
    lo_vec = plsc.load_gather(scv, [w * BAGS_PER_W + it])
    hi_vec = plsc.load_gather(scv, [w * BAGS_PER_W + 1 + it])
    for l in range(3):
        mrow = jnp.full((16,), NEG, jnp.float32)
        srow = jnp.zeros((16,), jnp.float32)
        for b in range(BAGS_PER_W):
            sel = it == b
            start = jnp.max(jnp.where(sel, lo_vec, -2147483647))
            end = jnp.max(jnp.where(sel, hi_vec, -2147483647))
            nch = (end - start + 15) // 16

            def max_step(ci, acc):
                idx = start + ci * 16 + it
                v = plsc.load_gather(lgv, [l * N + jnp.minimum(idx, N - 1)])
                return jnp.maximum(acc, jnp.where(idx < end, v, NEG))

            mvec = lax.fori_loop(0, nch, max_step, jnp.full((16,), NEG, jnp.float32))
            m = jnp.max(mvec)

            def sum_step(ci, acc):
                idx = start + ci * 16 + it
                v = plsc.load_gather(lgv, [l * N + jnp.minimum(idx, N - 1)])
                return acc + jnp.where(idx < end, jnp.exp(v - m), 0.0)

            svec = lax.fori_loop(0, nch, sum_step, jnp.zeros((16,), jnp.float32))
            s = jnp.sum(svec)
            mrow = jnp.where(sel, m, mrow)
            srow = jnp.where(sel, s, srow)
        mbuf[...] = mrow
        sbuf[...] = srow
        pltpu.sync_copy(mbuf, m_out.at[pl.ds(l * B + w * BAGS_PER_W, BAGS_PER_W)])
        pltpu.sync_copy(sbuf, s_out.at[pl.ds(l * B + w * BAGS_PER_W, BAGS_PER_W)])


def _sc_stats(lg, scope_pad):
    kfn = pl.kernel(
        _stats_body,
        out_type=(
            jax.ShapeDtypeStruct((3 * B,), jnp.float32),
            jax.ShapeDtypeStruct((3 * B,), jnp.float32),
        ),
        mesh=plsc.VectorSubcoreMesh(core_axis_name="c", subcore_axis_name="s"),
        compiler_params=_SC_PARAMS,
        scratch_types=[
            pltpu.VMEM((3 * N,), jnp.float32),
            pltpu.VMEM((520,), jnp.int32),
            pltpu.VMEM((16,), jnp.float32),
            pltpu.VMEM((16,), jnp.float32),
        ],
    )
    return kfn(lg, scope_pad)


SB = 512
CBLK = 512


def _pool_body(x_ref, lg_ref, m_ref, s_ref, lo_ref, hi_ref, wr_ref, b_ref,
               out_ref, cat_ref, pr_ref):
    g = pl.program_id(0)
    NSB = N // SB

    @pl.when(g < NSB)
    def _():
        rows = g * SB + lax.broadcasted_iota(jnp.int32, (1, SB), 1)
        lo = lo_ref[...]
        hi = hi_ref[...]
        mask = (rows >= lo) & (rows < hi)
        for l in range(3):
            lg = lg_ref[l:l + 1, :]
            m = m_ref[l]
            s = s_ref[l]
            e = jnp.where(mask, jnp.exp(lg - m), 0.0)
            wmat = (e / jnp.maximum(s, 1e-20)).astype(jnp.bfloat16)
            xb = x_ref[:, l * D:(l + 1) * D].astype(jnp.bfloat16)
            contrib = lax.dot_general(wmat, xb, (((1,), (0,)), ((), ())),
                                      preferred_element_type=jnp.float32)

            @pl.when(g == 0)
            def _():
                out_ref[l] = contrib

            @pl.when(g != 0)
            def _():
                out_ref[l] = out_ref[l] + contrib

            @pl.when(g == NSB - 1)
            def _():
                cat_ref[:, l * D:(l + 1) * D] = out_ref[l]

    @pl.when(g >= NSB)
    def _():
        lt = cat_ref[...].astype(jnp.bfloat16)
        wrb = wr_ref[...].astype(jnp.bfloat16)
        acc = lax.dot_general(lt, wrb, (((1,), (1,)), ((), ())),
                              preferred_element_type=jnp.float32)
        pr_ref[...] = acc + b_ref[...]


def _tc_pool(x2, lg2, m3, s3, lo2, hi2, wr, bias2):
    nsb = N // SB
    return pl.pallas_call(
        _pool_body,
        grid=(nsb + C_FLAT // CBLK,),
        in_specs=[
            pl.BlockSpec((SB, 3 * D), lambda g: (jnp.minimum(g, N // SB - 1), 0)),
            pl.BlockSpec((3, SB), lambda g: (0, jnp.minimum(g, N // SB - 1))),
            pl.BlockSpec((3, B, 1), lambda g: (0, 0, 0)),
            pl.BlockSpec((3, B, 1), lambda g: (0, 0, 0)),
            pl.BlockSpec((B, 1), lambda g: (0, 0)),
            pl.BlockSpec((B, 1), lambda g: (0, 0)),
            pl.BlockSpec((CBLK, 3 * D), lambda g: (jnp.maximum(g - N // SB, 0), 0)),
            pl.BlockSpec((1, CBLK), lambda g: (0, jnp.maximum(g - N // SB, 0))),
        ],
        out_specs=[
            pl.BlockSpec((3, B, D), lambda g: (0, 0, 0)),
            pl.BlockSpec((B, 3 * D), lambda g: (0, 0)),
            pl.BlockSpec((B, CBLK), lambda g: (0, jnp.maximum(g - N // SB, 0))),
        ],
        out_shape=[
            jax.ShapeDtypeStruct((3, B, D), jnp.float32),
            jax.ShapeDtypeStruct((B, 3 * D), jnp.float32),
            jax.ShapeDtypeStruct((B, C_FLAT), jnp.float32),
        ],
    )(x2, lg2, m3, s3, lo2, hi2, wr, bias2)


def kernel(x, attention_query, scope, relation_weight, bias, attention_weight):
    x_flat = x.reshape(N * 3 * D)
    q3n = attention_query.astype(jnp.int32).reshape(3 * N)
    scope_pad = jnp.pad(scope.astype(jnp.int32), (0, 520 - B - 1))

    lg2 = _sc_logits(x_flat, q3n, attention_weight)
    m3, s3 = _sc_stats(lg2, scope_pad)

    lo2 = scope[:B].astype(jnp.int32).reshape(B, 1)
    hi2 = scope[1:].astype(jnp.int32).reshape(B, 1)
    layers, logits_total, probs = _tc_pool(
        x.reshape(N, 3 * D), lg2.reshape(3, N), m3.reshape(3, B, 1),
        s3.reshape(3, B, 1), lo2, hi2, relation_weight,
        bias.reshape(1, C_FLAT))
    return (layers, logits_total, probs)
